# baseline (speedup 1.0000x reference)
import functools

import jax
import jax.numpy as jnp
from jax.experimental import pallas as pl
from jax.experimental.pallas import tpu as pltpu

NUM_EXPERTS = 64
NUM_GROUPS = 8
GROUP_SIZE = NUM_EXPERTS // NUM_GROUPS
TOP_GROUPS = 4
TOPK = 8
TOKEN_BLOCK = 1024
KSPLIT = 2

_NEG = float("-inf")


def _router_kernel(x_ref, w_ref, b_ref, ts_ref, idx_ref, cnt_ref, acc_ref):
    i = pl.program_id(0)
    k = pl.program_id(1)
    x = x_ref[...]
    dk = x.shape[1]
    w = w_ref[:, pl.ds(k * dk, dk)]
    partial = jax.lax.dot_general(
        x, w, (((1,), (1,)), ((), ())), preferred_element_type=jnp.float32
    )

    @pl.when(k == 0)
    def _():
        acc_ref[...] = partial

    @pl.when(k == KSPLIT - 1)
    def _():
        lt = (acc_ref[...] + partial).T
        s = jax.nn.sigmoid(lt)
        sfc = s + b_ref[...]
        tb = s.shape[1]

        li8 = jax.lax.broadcasted_iota(jnp.int32, (GROUP_SIZE, tb), 0)
        gs_rows = []
        for g in range(NUM_GROUPS):
            slab = sfc[g * GROUP_SIZE:(g + 1) * GROUP_SIZE, :]
            m1 = jnp.max(slab, axis=0, keepdims=True)
            i1 = jnp.min(jnp.where(slab == m1, li8, GROUP_SIZE), axis=0,
                         keepdims=True)
            m2 = jnp.max(jnp.where(li8 == i1, _NEG, slab), axis=0,
                         keepdims=True)
            gs_rows.append(m1 + m2)
        gs = jnp.concatenate(gs_rows, axis=0)

        iota_g = jax.lax.broadcasted_iota(jnp.int32, (NUM_GROUPS, tb), 0)
        keep = jnp.zeros((NUM_GROUPS, tb), jnp.bool_)
        gm = gs
        for _ in range(TOP_GROUPS):
            mg = jnp.max(gm, axis=0, keepdims=True)
            gi = jnp.min(jnp.where(gm == mg, iota_g, NUM_GROUPS), axis=0,
                         keepdims=True)
            hit_g = iota_g == gi
            keep = keep | hit_g
            gm = jnp.where(hit_g, _NEG, gm)

        masked = jnp.concatenate(
            [jnp.where(keep[g:g + 1, :],
                       sfc[g * GROUP_SIZE:(g + 1) * GROUP_SIZE, :], _NEG)
             for g in range(NUM_GROUPS)], axis=0)

        iota_e = jax.lax.broadcasted_iota(jnp.int32, (NUM_EXPERTS, tb), 0)
        vals, idxs = [], []
        selcnt = jnp.zeros((NUM_EXPERTS, tb), jnp.float32)
        msk = masked
        for _ in range(TOPK):
            m = jnp.max(msk, axis=0, keepdims=True)
            e = jnp.min(jnp.where(msk == m, iota_e, NUM_EXPERTS), axis=0,
                        keepdims=True)
            hit = iota_e == e
            vals.append(jnp.sum(jnp.where(hit, s, 0.0), axis=0, keepdims=True))
            idxs.append(e)
            selcnt = selcnt + hit.astype(jnp.float32)
            msk = jnp.where(hit, _NEG, msk)
        vt = jnp.concatenate(vals, axis=0)
        it = jnp.concatenate(idxs, axis=0)

        denom = jnp.sum(vt, axis=0, keepdims=True) + 1e-20
        ts_ref[...] = (vt / denom).T
        idx_ref[...] = it.T

        blk_cnt = jnp.sum(selcnt, axis=1, keepdims=True).astype(jnp.int32)

        @pl.when(i == 0)
        def _():
            cnt_ref[...] = blk_cnt

        @pl.when(i != 0)
        def _():
            cnt_ref[...] = cnt_ref[...] + blk_cnt


@functools.partial(jax.jit, static_argnames=())
def kernel(x, expert_bias, W):
    n, dim = x.shape
    b = expert_bias.reshape(NUM_EXPERTS, 1)
    grid = (n // TOKEN_BLOCK, KSPLIT)
    ts, idx, cnt = pl.pallas_call(
        _router_kernel,
        grid=grid,
        in_specs=[
            pl.BlockSpec((TOKEN_BLOCK, dim // KSPLIT), lambda i, k: (i, k)),
            pl.BlockSpec((NUM_EXPERTS, dim), lambda i, k: (0, 0)),
            pl.BlockSpec((NUM_EXPERTS, 1), lambda i, k: (0, 0)),
        ],
        out_specs=[
            pl.BlockSpec((TOKEN_BLOCK, TOPK), lambda i, k: (i, 0)),
            pl.BlockSpec((TOKEN_BLOCK, TOPK), lambda i, k: (i, 0)),
            pl.BlockSpec((NUM_EXPERTS, 1), lambda i, k: (0, 0)),
        ],
        out_shape=[
            jax.ShapeDtypeStruct((n, TOPK), jnp.float32),
            jax.ShapeDtypeStruct((n, TOPK), jnp.int32),
            jax.ShapeDtypeStruct((NUM_EXPERTS, 1), jnp.int32),
        ],
        scratch_shapes=[pltpu.VMEM((TOKEN_BLOCK, NUM_EXPERTS), jnp.float32)],
        compiler_params=pltpu.CompilerParams(
            dimension_semantics=("arbitrary", "arbitrary"),
        ),
    )(x, W, b)
    return ts, idx, cnt.reshape(NUM_EXPERTS)


# speedup vs baseline: 1.2613x; 1.2613x over previous
import functools

import jax
import jax.numpy as jnp
from jax.experimental import pallas as pl
from jax.experimental.pallas import tpu as pltpu

NUM_EXPERTS = 64
NUM_GROUPS = 8
GROUP_SIZE = NUM_EXPERTS // NUM_GROUPS
TOP_GROUPS = 4
TOPK = 8
TOKEN_BLOCK = 1024

_NEG = float("-inf")


def _router_kernel(x_ref, w_ref, b_ref, ts_ref, idx_ref, cnt_ref):
    i = pl.program_id(0)
    x = x_ref[...]
    w = w_ref[...]
    lt = jax.lax.dot_general(
        x, w, (((1,), (1,)), ((), ())), preferred_element_type=jnp.float32
    ).T
    s = jax.nn.sigmoid(lt)
    sfc = s + b_ref[...]
    tb = s.shape[1]

    li8 = jax.lax.broadcasted_iota(jnp.int32, (GROUP_SIZE, tb), 0)
    gs_rows = []
    for g in range(NUM_GROUPS):
        slab = sfc[g * GROUP_SIZE:(g + 1) * GROUP_SIZE, :]
        m1 = jnp.max(slab, axis=0, keepdims=True)
        i1 = jnp.min(jnp.where(slab == m1, li8, GROUP_SIZE), axis=0,
                     keepdims=True)
        m2 = jnp.max(jnp.where(li8 == i1, _NEG, slab), axis=0, keepdims=True)
        gs_rows.append(m1 + m2)
    gs = jnp.concatenate(gs_rows, axis=0)

    iota_g = jax.lax.broadcasted_iota(jnp.int32, (NUM_GROUPS, tb), 0)
    keep = jnp.zeros((NUM_GROUPS, tb), jnp.bool_)
    gm = gs
    for _ in range(TOP_GROUPS):
        mg = jnp.max(gm, axis=0, keepdims=True)
        gi = jnp.min(jnp.where(gm == mg, iota_g, NUM_GROUPS), axis=0,
                     keepdims=True)
        hit_g = iota_g == gi
        keep = keep | hit_g
        gm = jnp.where(hit_g, _NEG, gm)

    masked = jnp.concatenate(
        [jnp.where(keep[g:g + 1, :], sfc[g * GROUP_SIZE:(g + 1) * GROUP_SIZE, :], _NEG)
         for g in range(NUM_GROUPS)], axis=0)

    iota_e = jax.lax.broadcasted_iota(jnp.int32, (NUM_EXPERTS, tb), 0)
    vals, idxs = [], []
    selcnt = jnp.zeros((NUM_EXPERTS, tb), jnp.float32)
    for _ in range(TOPK):
        m = jnp.max(masked, axis=0, keepdims=True)
        e = jnp.min(jnp.where(masked == m, iota_e, NUM_EXPERTS), axis=0,
                    keepdims=True)
        hit = iota_e == e
        vals.append(jnp.sum(jnp.where(hit, s, 0.0), axis=0, keepdims=True))
        idxs.append(e)
        selcnt = selcnt + hit.astype(jnp.float32)
        masked = jnp.where(hit, _NEG, masked)
    vt = jnp.concatenate(vals, axis=0)
    it = jnp.concatenate(idxs, axis=0)

    denom = jnp.sum(vt, axis=0, keepdims=True) + 1e-20
    ts_ref[...] = (vt / denom).T
    idx_ref[...] = it.T

    blk_cnt = jnp.sum(selcnt, axis=1, keepdims=True).astype(jnp.int32)

    @pl.when(i == 0)
    def _():
        cnt_ref[...] = blk_cnt

    @pl.when(i != 0)
    def _():
        cnt_ref[...] = cnt_ref[...] + blk_cnt


@functools.partial(jax.jit, static_argnames=())
def kernel(x, expert_bias, W):
    n, dim = x.shape
    b = expert_bias.reshape(NUM_EXPERTS, 1)
    grid = (n // TOKEN_BLOCK,)
    ts, idx, cnt = pl.pallas_call(
        _router_kernel,
        grid=grid,
        in_specs=[
            pl.BlockSpec((TOKEN_BLOCK, dim), lambda i: (i, 0)),
            pl.BlockSpec((NUM_EXPERTS, dim), lambda i: (0, 0)),
            pl.BlockSpec((NUM_EXPERTS, 1), lambda i: (0, 0)),
        ],
        out_specs=[
            pl.BlockSpec((TOKEN_BLOCK, TOPK), lambda i: (i, 0)),
            pl.BlockSpec((TOKEN_BLOCK, TOPK), lambda i: (i, 0)),
            pl.BlockSpec((NUM_EXPERTS, 1), lambda i: (0, 0)),
        ],
        out_shape=[
            jax.ShapeDtypeStruct((n, TOPK), jnp.float32),
            jax.ShapeDtypeStruct((n, TOPK), jnp.int32),
            jax.ShapeDtypeStruct((NUM_EXPERTS, 1), jnp.int32),
        ],
        compiler_params=pltpu.CompilerParams(
            dimension_semantics=("arbitrary",),
        ),
    )(x, W, b)
    return ts, idx, cnt.reshape(NUM_EXPERTS)
